# Initial kernel scaffold; baseline (speedup 1.0000x reference)
import jax
import jax.numpy as jnp
from jax.experimental import pallas as pl


def kernel(x, edge_index, W1_0, b1_0, W2_0, b2_0, W1_1, b1_1, W2_1, b2_1, W1_2, b1_2, W2_2, b2_2):
    raise NotImplementedError("write your pallas kernel here")


# speedup vs baseline: 8.1441x; 8.1441x over previous
import functools

import jax
import jax.numpy as jnp
from jax import lax
from jax.experimental import pallas as pl
from jax.experimental.pallas import tpu as pltpu
from jax.experimental.pallas import tpu_sc as plsc

N = 10000
D = 128
E = 320000
HID = 256

NC = 2
NS = 16
NW = NC * NS
EDGES_PER_TILE = E // NW
CH = 80
NCH = EDGES_PER_TILE // CH
PAD_N = 10240
ROWS_PER_TILE = PAD_N // NS
ZROWS = 128


def _sc_aggregate(h, src, dst, zeros):
    mesh = plsc.VectorSubcoreMesh(core_axis_name="c", subcore_axis_name="s")

    @functools.partial(
        pl.kernel,
        mesh=mesh,
        out_type=jax.ShapeDtypeStruct((NC, PAD_N, D), jnp.float32),
        scratch_types=[
            pltpu.VMEM((CH,), jnp.int32),
            pltpu.VMEM((CH,), jnp.int32),
            pltpu.VMEM((CH,), jnp.int32),
            pltpu.VMEM((CH,), jnp.int32),
            pltpu.VMEM((CH, D), jnp.float32),
            pltpu.VMEM((CH, D), jnp.float32),
            pltpu.VMEM_SHARED((PAD_N, D), jnp.float32),
            pltpu.SemaphoreType.DMA,
            pltpu.SemaphoreType.DMA,
            pltpu.SemaphoreType.DMA,
            pltpu.SemaphoreType.DMA,
        ],
    )
    def agg_kernel(h_hbm, src_hbm, dst_hbm, z_hbm, out_hbm,
                   src_a, src_b, dst_a, dst_b, rows_a, rows_b, acc_sh,
                   sem_a, sem_b, sem_ia, sem_ib):
        cid = lax.axis_index("c")
        sid = lax.axis_index("s")
        base = (cid * NS + sid) * EDGES_PER_TILE
        row0 = sid * ROWS_PER_TILE

        def idx_start(j, sbuf, dbuf, sem):
            pltpu.async_copy(src_hbm.at[pl.ds(base + j * CH, CH)], sbuf, sem)
            pltpu.async_copy(dst_hbm.at[pl.ds(base + j * CH, CH)], dbuf, sem)

        def idx_wait(j, sbuf, dbuf, sem):
            pltpu.make_async_copy(src_hbm.at[pl.ds(base + j * CH, CH)], sbuf, sem).wait()
            pltpu.make_async_copy(dst_hbm.at[pl.ds(base + j * CH, CH)], dbuf, sem).wait()

        def gather_start(sbuf, buf, sem):
            pltpu.async_copy(h_hbm.at[sbuf], buf, sem)

        def gather_wait(sbuf, buf, sem):
            pltpu.make_async_copy(h_hbm.at[sbuf], buf, sem).wait()

        def scatter(buf, dbuf):
            pltpu.sync_copy(buf, acc_sh.at[dbuf], add=True)

        idx_start(0, src_a, dst_a, sem_ia)
        idx_wait(0, src_a, dst_a, sem_ia)
        gather_start(src_a, rows_a, sem_a)
        idx_start(1, src_b, dst_b, sem_ib)
        pltpu.sync_copy(z_hbm.at[pl.ds(row0, ROWS_PER_TILE)],
                        acc_sh.at[pl.ds(row0, ROWS_PER_TILE)])
        plsc.subcore_barrier()

        @pl.loop(0, NCH - 1, step=2)
        def _(j):
            gather_wait(src_a, rows_a, sem_a)
            idx_wait(j + 1, src_b, dst_b, sem_ib)
            gather_start(src_b, rows_b, sem_b)
            scatter(rows_a, dst_a)

            @pl.when(j + 2 < NCH)
            def _():
                idx_start(j + 2, src_a, dst_a, sem_ia)

            gather_wait(src_b, rows_b, sem_b)

            @pl.when(j + 2 < NCH)
            def _():
                idx_wait(j + 2, src_a, dst_a, sem_ia)
                gather_start(src_a, rows_a, sem_a)

            scatter(rows_b, dst_b)

            @pl.when(j + 3 < NCH)
            def _():
                idx_start(j + 3, src_b, dst_b, sem_ib)

        gather_wait(src_a, rows_a, sem_a)
        scatter(rows_a, dst_a)

        plsc.subcore_barrier()

        pltpu.sync_copy(acc_sh.at[pl.ds(row0, ROWS_PER_TILE)],
                        out_hbm.at[cid, pl.ds(row0, ROWS_PER_TILE)])

    return agg_kernel(h, src, dst, zeros)


def _tc_mlp(h, agg0, agg1, W1, b1, W2, b2, act):
    BN = 1000

    def mlp_kernel(h_ref, a0_ref, a1_ref, W1_ref, b1_ref, W2_ref, b2_ref, o_ref):
        z = h_ref[...] + a0_ref[...] + a1_ref[...]
        t = jnp.dot(z, W1_ref[...], preferred_element_type=jnp.float32)
        t = t + b1_ref[...]
        t = jnp.where(t > 0, t, 0.2 * t)
        o = jnp.dot(t, W2_ref[...], preferred_element_type=jnp.float32)
        o = o + b2_ref[...]
        if act:
            o = jnp.where(o > 0, o, 0.2 * o)
        o_ref[...] = o

    return pl.pallas_call(
        mlp_kernel,
        grid=(N // BN,),
        in_specs=[
            pl.BlockSpec((BN, D), lambda i: (i, 0)),
            pl.BlockSpec((BN, D), lambda i: (i, 0)),
            pl.BlockSpec((BN, D), lambda i: (i, 0)),
            pl.BlockSpec((D, HID), lambda i: (0, 0)),
            pl.BlockSpec((1, HID), lambda i: (0, 0)),
            pl.BlockSpec((HID, D), lambda i: (0, 0)),
            pl.BlockSpec((1, D), lambda i: (0, 0)),
        ],
        out_specs=pl.BlockSpec((BN, D), lambda i: (i, 0)),
        out_shape=jax.ShapeDtypeStruct((N, D), jnp.float32),
    )(h, agg0, agg1, W1, b1.reshape(1, HID), W2, b2.reshape(1, D))


def kernel(x, edge_index,
           W1_0, b1_0, W2_0, b2_0,
           W1_1, b1_1, W2_1, b2_1,
           W1_2, b1_2, W2_2, b2_2):
    src = edge_index[0]
    dst = edge_index[1]
    zeros = jnp.zeros((PAD_N, D), jnp.float32)
    params = [(W1_0, b1_0, W2_0, b2_0),
              (W1_1, b1_1, W2_1, b2_1),
              (W1_2, b1_2, W2_2, b2_2)]
    h = x
    for l in range(3):
        agg = _sc_aggregate(h, src, dst, zeros)
        h = _tc_mlp(h, agg[0], agg[1], *params[l], act=(l < 2))
    return h
